# Initial kernel scaffold; baseline (speedup 1.0000x reference)
import jax
import jax.numpy as jnp
from jax.experimental import pallas as pl


def kernel(x, emb_weight):
    raise NotImplementedError("write your pallas kernel here")


# speedup vs baseline: 1.9526x; 1.9526x over previous
import jax
import jax.numpy as jnp
from jax.experimental import pallas as pl

MAX_POSITIONS = 8192
SEQ_BLOCK = 512


def _add_kernel(x_ref, emb_ref, out_ref):
    out_ref[...] = x_ref[...] + emb_ref[...][None, :, :]


def kernel(x, emb_weight):
    batch, seq_len, dim = x.shape
    offset_blocks = (emb_weight.shape[0] - seq_len) // SEQ_BLOCK
    num_blocks = seq_len // SEQ_BLOCK
    return pl.pallas_call(
        _add_kernel,
        grid=(num_blocks,),
        in_specs=[
            pl.BlockSpec((batch, SEQ_BLOCK, dim), lambda s: (0, s, 0)),
            pl.BlockSpec((SEQ_BLOCK, dim), lambda s: (s + offset_blocks, 0)),
        ],
        out_specs=pl.BlockSpec((batch, SEQ_BLOCK, dim), lambda s: (0, s, 0)),
        out_shape=jax.ShapeDtypeStruct(x.shape, x.dtype),
    )(x, emb_weight)
